# baseline (speedup 1.0000x reference)
import functools

import jax
import jax.numpy as jnp
from jax import lax
from jax.experimental import pallas as pl
from jax.experimental.pallas import tpu as pltpu

_EPS = 1e-5


def _upconv_kernel(x_ref, w_ref, b_ref, o_ref):
    H, _, W, K2 = o_ref.shape
    x = x_ref[0].astype(jnp.bfloat16)
    y = lax.dot_general(x, w_ref[...], (((0,), (0,)), ((), ())),
                        preferred_element_type=jnp.float32)
    y = y + b_ref[...]
    o_ref[:, 0, :, :] = y[:, :K2].reshape(H, W, K2).astype(o_ref.dtype)
    o_ref[:, 1, :, :] = y[:, K2:].reshape(H, W, K2).astype(o_ref.dtype)


def _double_conv_kernel(xa_ref, xb_ref, w1s_ref, b1_ref,
                        w2s_ref, b2_ref, o_ref,
                        sbuf1, sbuf2, *, oh, ow):
    M = xa_ref.shape[2]
    C = xa_ref.shape[1]

    flat = lax.broadcasted_iota(jnp.int32, (M, 1), 0)
    col = flat & (ow - 1) if (ow & (ow - 1)) == 0 else flat % ow
    m_l = col >= 1
    m_r = col < (ow - 1)

    def shifted(x):
        z = jnp.zeros_like(x)
        return (jnp.where(m_l, pltpu.roll(x, 1, axis=0), z),
                jnp.where(m_r, pltpu.roll(x, M - 1, axis=0), z))

    def conv(sbuf, pieces, ws_ref, nlanes):
        sbuf[0:ow, :] = jnp.zeros((ow, nlanes), jnp.bfloat16)
        sbuf[ow + M:, :] = jnp.zeros((ow, nlanes), jnp.bfloat16)
        for k, piece in enumerate(pieces):
            sbuf[pl.ds(ow, M), k * C:(k + 1) * C] = piece
        acc = jnp.dot(sbuf[pl.ds(0, M), :], ws_ref[0],
                      preferred_element_type=jnp.float32)
        acc += jnp.dot(sbuf[pl.ds(ow, M), :], ws_ref[1],
                       preferred_element_type=jnp.float32)
        acc += jnp.dot(sbuf[pl.ds(2 * ow, M), :], ws_ref[2],
                       preferred_element_type=jnp.float32)
        return acc

    xa = jnp.transpose(xa_ref[0], (1, 0)).astype(jnp.bfloat16)
    xb = xb_ref[0]
    xa_m, xa_p = shifted(xa)
    xb_m, xb_p = shifted(xb)
    acc1 = conv(sbuf1, (xa_m, xa, xa_p, xb_m, xb, xb_p), w1s_ref, 6 * C)
    h1 = jnp.maximum(acc1 + b1_ref[...], 0.0).astype(jnp.bfloat16)

    h1_m, h1_p = shifted(h1)
    acc2 = conv(sbuf2, (h1_m, h1, h1_p), w2s_ref, 3 * b1_ref.shape[1])
    y2 = jnp.maximum(acc2 + b2_ref[...], 0.0)
    o_ref[0] = jnp.transpose(y2, (1, 0))


def _fold_bn(conv_b, gamma, beta, rmean, rvar):
    scale = gamma / jnp.sqrt(rvar + _EPS)
    bias = beta + scale * (conv_b - rmean)
    return scale, bias


def kernel(x1, x2, up_w, up_b, c1_w, c1_b, bn1_g, bn1_b, bn1_m, bn1_v,
           c2_w, c2_b, bn2_g, bn2_b, bn2_m, bn2_v):
    N, C, H1, W1 = x1.shape
    oh, ow = 2 * H1, 2 * W1
    M = oh * ow
    Cmid = c1_w.shape[0]
    Cout = c2_w.shape[0]

    x1f = x1.reshape(N, C, H1 * W1)
    x2f = x2.reshape(N, C, M)

    wup = jnp.transpose(up_w, (0, 2, 3, 1)).reshape(C, 4 * C)
    wup = wup.astype(jnp.bfloat16)
    bup = jnp.tile(up_b, 4)[None, :]

    up_out = pl.pallas_call(
        _upconv_kernel,
        out_shape=jax.ShapeDtypeStruct((N * H1, 2, W1, 2 * C),
                                       jnp.bfloat16),
        grid=(N,),
        in_specs=[
            pl.BlockSpec((1, C, H1 * W1), lambda n: (n, 0, 0)),
            pl.BlockSpec((C, 4 * C), lambda n: (0, 0)),
            pl.BlockSpec((1, 4 * C), lambda n: (0, 0)),
        ],
        out_specs=pl.BlockSpec((H1, 2, W1, 2 * C), lambda n: (n, 0, 0, 0)),
        compiler_params=pltpu.CompilerParams(
            dimension_semantics=("parallel",),
            vmem_limit_bytes=64 * 1024 * 1024,
        ),
    )(x1f, wup, bup)
    x1u = up_out.reshape(N, M, C)

    s1, b1 = _fold_bn(c1_b, bn1_g, bn1_b, bn1_m, bn1_v)
    s2, b2 = _fold_bn(c2_b, bn2_g, bn2_b, bn2_m, bn2_v)

    w1 = jnp.transpose(c1_w, (2, 3, 1, 0)).reshape(3, 3, 2 * C, Cmid)
    w1s = jnp.concatenate(
        [w1[:, 0, :C], w1[:, 1, :C], w1[:, 2, :C],
         w1[:, 0, C:], w1[:, 1, C:], w1[:, 2, C:]], axis=1)
    w1s = (w1s * s1[None, None, :]).astype(jnp.bfloat16)
    w2 = jnp.transpose(c2_w, (2, 3, 1, 0)).reshape(3, 3, Cmid, Cout)
    w2s = jnp.concatenate([w2[:, 0], w2[:, 1], w2[:, 2]], axis=1)
    w2s = (w2s * s2[None, None, :]).astype(jnp.bfloat16)

    body = functools.partial(_double_conv_kernel, oh=oh, ow=ow)
    out = pl.pallas_call(
        body,
        out_shape=jax.ShapeDtypeStruct((N, Cout, M), jnp.float32),
        grid=(N,),
        in_specs=[
            pl.BlockSpec((1, C, M), lambda n: (n, 0, 0)),
            pl.BlockSpec((1, M, C), lambda n: (n, 0, 0)),
            pl.BlockSpec((3, 6 * C, Cmid), lambda n: (0, 0, 0)),
            pl.BlockSpec((1, Cmid), lambda n: (0, 0)),
            pl.BlockSpec((3, 3 * Cmid, Cout), lambda n: (0, 0, 0)),
            pl.BlockSpec((1, Cout), lambda n: (0, 0)),
        ],
        out_specs=pl.BlockSpec((1, Cout, M), lambda n: (n, 0, 0)),
        scratch_shapes=[
            pltpu.VMEM((M + 2 * ow, 6 * C), jnp.bfloat16),
            pltpu.VMEM((M + 2 * ow, 3 * Cmid), jnp.bfloat16),
        ],
        compiler_params=pltpu.CompilerParams(
            dimension_semantics=("parallel",),
            vmem_limit_bytes=64 * 1024 * 1024,
        ),
    )(x2f, x1u, w1s, b1[None, :], w2s, b2[None, :])

    return out.reshape(N, Cout, oh, ow)


# speedup vs baseline: 1.0757x; 1.0757x over previous
import functools

import jax
import jax.numpy as jnp
from jax import lax
from jax.experimental import pallas as pl
from jax.experimental.pallas import tpu as pltpu

_EPS = 1e-5
_UPB = 4
_DCB = 2


def _upconv_kernel(x_ref, w_ref, b_ref, o_ref):
    B = x_ref.shape[0]
    H = o_ref.shape[0] // B
    W, K2 = o_ref.shape[2], o_ref.shape[3]
    for i in range(B):
        x = x_ref[i].astype(jnp.bfloat16)
        y = lax.dot_general(x, w_ref[...], (((0,), (0,)), ((), ())),
                            preferred_element_type=jnp.float32)
        y = y + b_ref[...]
        o_ref[i * H:(i + 1) * H, 0, :, :] = (
            y[:, :K2].reshape(H, W, K2).astype(o_ref.dtype))
        o_ref[i * H:(i + 1) * H, 1, :, :] = (
            y[:, K2:].reshape(H, W, K2).astype(o_ref.dtype))


def _double_conv_kernel(xa_ref, xb_ref, w1s_ref, b1_ref,
                        w2s_ref, b2_ref, o_ref,
                        sb1a, sb2a, sb1b, sb2b, *, oh, ow):
    M = xa_ref.shape[2]
    C = xa_ref.shape[1]
    Cmid = b1_ref.shape[1]

    flat = lax.broadcasted_iota(jnp.int32, (M, 1), 0)
    col = flat & (ow - 1) if (ow & (ow - 1)) == 0 else flat % ow
    m_l = col >= 1
    m_r = col < (ow - 1)

    def shifted(x):
        z = jnp.zeros_like(x)
        return (jnp.where(m_l, pltpu.roll(x, 1, axis=0), z),
                jnp.where(m_r, pltpu.roll(x, M - 1, axis=0), z))

    def conv(sbuf, pieces, ws_ref, nlanes):
        sbuf[0:ow, :] = jnp.zeros((ow, nlanes), jnp.bfloat16)
        sbuf[ow + M:, :] = jnp.zeros((ow, nlanes), jnp.bfloat16)
        for k, piece in enumerate(pieces):
            sbuf[pl.ds(ow, M), k * C:(k + 1) * C] = piece
        acc = jnp.dot(sbuf[pl.ds(0, M), :], ws_ref[0],
                      preferred_element_type=jnp.float32)
        acc += jnp.dot(sbuf[pl.ds(ow, M), :], ws_ref[1],
                       preferred_element_type=jnp.float32)
        acc += jnp.dot(sbuf[pl.ds(2 * ow, M), :], ws_ref[2],
                       preferred_element_type=jnp.float32)
        return acc

    for i, (sb1, sb2) in enumerate(((sb1a, sb2a), (sb1b, sb2b))):
        xa = jnp.transpose(xa_ref[i], (1, 0)).astype(jnp.bfloat16)
        xb = xb_ref[i]
        xa_m, xa_p = shifted(xa)
        xb_m, xb_p = shifted(xb)
        acc1 = conv(sb1, (xa_m, xa, xa_p, xb_m, xb, xb_p), w1s_ref, 6 * C)
        h1 = jnp.maximum(acc1 + b1_ref[...], 0.0).astype(jnp.bfloat16)

        h1_m, h1_p = shifted(h1)
        acc2 = conv(sb2, (h1_m, h1, h1_p), w2s_ref, 3 * Cmid)
        y2 = jnp.maximum(acc2 + b2_ref[...], 0.0)
        o_ref[i] = jnp.transpose(y2, (1, 0))


def _fold_bn(conv_b, gamma, beta, rmean, rvar):
    scale = gamma / jnp.sqrt(rvar + _EPS)
    bias = beta + scale * (conv_b - rmean)
    return scale, bias


def kernel(x1, x2, up_w, up_b, c1_w, c1_b, bn1_g, bn1_b, bn1_m, bn1_v,
           c2_w, c2_b, bn2_g, bn2_b, bn2_m, bn2_v):
    N, C, H1, W1 = x1.shape
    oh, ow = 2 * H1, 2 * W1
    M = oh * ow
    Cmid = c1_w.shape[0]
    Cout = c2_w.shape[0]

    x1f = x1.reshape(N, C, H1 * W1)
    x2f = x2.reshape(N, C, M)

    wup = jnp.transpose(up_w, (0, 2, 3, 1)).reshape(C, 4 * C)
    wup = wup.astype(jnp.bfloat16)
    bup = jnp.tile(up_b, 4)[None, :]

    up_out = pl.pallas_call(
        _upconv_kernel,
        out_shape=jax.ShapeDtypeStruct((N * H1, 2, W1, 2 * C),
                                       jnp.bfloat16),
        grid=(N // _UPB,),
        in_specs=[
            pl.BlockSpec((_UPB, C, H1 * W1), lambda n: (n, 0, 0)),
            pl.BlockSpec((C, 4 * C), lambda n: (0, 0)),
            pl.BlockSpec((1, 4 * C), lambda n: (0, 0)),
        ],
        out_specs=pl.BlockSpec((_UPB * H1, 2, W1, 2 * C),
                               lambda n: (n, 0, 0, 0)),
        compiler_params=pltpu.CompilerParams(
            dimension_semantics=("parallel",),
            vmem_limit_bytes=64 * 1024 * 1024,
        ),
    )(x1f, wup, bup)
    x1u = up_out.reshape(N, M, C)

    s1, b1 = _fold_bn(c1_b, bn1_g, bn1_b, bn1_m, bn1_v)
    s2, b2 = _fold_bn(c2_b, bn2_g, bn2_b, bn2_m, bn2_v)

    w1 = jnp.transpose(c1_w, (2, 3, 1, 0)).reshape(3, 3, 2 * C, Cmid)
    w1s = jnp.concatenate(
        [w1[:, 0, :C], w1[:, 1, :C], w1[:, 2, :C],
         w1[:, 0, C:], w1[:, 1, C:], w1[:, 2, C:]], axis=1)
    w1s = (w1s * s1[None, None, :]).astype(jnp.bfloat16)
    w2 = jnp.transpose(c2_w, (2, 3, 1, 0)).reshape(3, 3, Cmid, Cout)
    w2s = jnp.concatenate([w2[:, 0], w2[:, 1], w2[:, 2]], axis=1)
    w2s = (w2s * s2[None, None, :]).astype(jnp.bfloat16)

    body = functools.partial(_double_conv_kernel, oh=oh, ow=ow)
    out = pl.pallas_call(
        body,
        out_shape=jax.ShapeDtypeStruct((N, Cout, M), jnp.float32),
        grid=(N // _DCB,),
        in_specs=[
            pl.BlockSpec((_DCB, C, M), lambda n: (n, 0, 0)),
            pl.BlockSpec((_DCB, M, C), lambda n: (n, 0, 0)),
            pl.BlockSpec((3, 6 * C, Cmid), lambda n: (0, 0, 0)),
            pl.BlockSpec((1, Cmid), lambda n: (0, 0)),
            pl.BlockSpec((3, 3 * Cmid, Cout), lambda n: (0, 0, 0)),
            pl.BlockSpec((1, Cout), lambda n: (0, 0)),
        ],
        out_specs=pl.BlockSpec((_DCB, Cout, M), lambda n: (n, 0, 0)),
        scratch_shapes=[
            pltpu.VMEM((M + 2 * ow, 6 * C), jnp.bfloat16),
            pltpu.VMEM((M + 2 * ow, 3 * Cmid), jnp.bfloat16),
            pltpu.VMEM((M + 2 * ow, 6 * C), jnp.bfloat16),
            pltpu.VMEM((M + 2 * ow, 3 * Cmid), jnp.bfloat16),
        ],
        compiler_params=pltpu.CompilerParams(
            dimension_semantics=("parallel",),
            vmem_limit_bytes=64 * 1024 * 1024,
        ),
    )(x2f, x1u, w1s, b1[None, :], w2s, b2[None, :])

    return out.reshape(N, Cout, oh, ow)


# speedup vs baseline: 1.2538x; 1.1656x over previous
import functools

import jax
import jax.numpy as jnp
from jax import lax
from jax.experimental import pallas as pl
from jax.experimental.pallas import tpu as pltpu

_EPS = 1e-5
_UPB = 4
_DCB = 2


def _upconv_kernel(x_ref, w_ref, b_ref, o_ref):
    B = x_ref.shape[0]
    H = o_ref.shape[0] // B
    W, K2 = o_ref.shape[2], o_ref.shape[3]
    for i in range(B):
        x = x_ref[i].astype(jnp.bfloat16)
        y = lax.dot_general(x, w_ref[...], (((0,), (0,)), ((), ())),
                            preferred_element_type=jnp.float32)
        y = y + b_ref[...]
        o_ref[i * H:(i + 1) * H, 0, :, :] = (
            y[:, :K2].reshape(H, W, K2).astype(o_ref.dtype))
        o_ref[i * H:(i + 1) * H, 1, :, :] = (
            y[:, K2:].reshape(H, W, K2).astype(o_ref.dtype))


def _double_conv_kernel(xa_ref, xb_ref, w1s_ref, b1_ref,
                        w2s_ref, b2_ref, o_ref,
                        sb1a, sb2a, sb1b, sb2b, *, oh, ow):
    M = oh * ow
    C = xa_ref.shape[1]
    Cmid = b1_ref.shape[1]

    flat = lax.broadcasted_iota(jnp.int32, (M, 1), 0)
    col = flat & (ow - 1) if (ow & (ow - 1)) == 0 else flat % ow
    m_l = col >= 1
    m_r = col < (ow - 1)

    def shifted(x):
        z = jnp.zeros_like(x)
        return (jnp.where(m_l, pltpu.roll(x, 1, axis=0), z),
                jnp.where(m_r, pltpu.roll(x, M - 1, axis=0), z))

    def conv(sbuf, pieces, ws_ref, nlanes):
        sbuf[0:ow, :] = jnp.zeros((ow, nlanes), jnp.bfloat16)
        sbuf[ow + M:, :] = jnp.zeros((ow, nlanes), jnp.bfloat16)
        for k, piece in enumerate(pieces):
            sbuf[pl.ds(ow, M), k * C:(k + 1) * C] = piece
        acc = jnp.dot(sbuf[pl.ds(0, M), :], ws_ref[0],
                      preferred_element_type=jnp.float32)
        acc += jnp.dot(sbuf[pl.ds(ow, M), :], ws_ref[1],
                       preferred_element_type=jnp.float32)
        acc += jnp.dot(sbuf[pl.ds(2 * ow, M), :], ws_ref[2],
                       preferred_element_type=jnp.float32)
        return acc

    for i, (sb1, sb2) in enumerate(((sb1a, sb2a), (sb1b, sb2b))):
        xa = jnp.transpose(xa_ref[i], (1, 2, 0)).reshape(M, C)
        xa = xa.astype(jnp.bfloat16)
        xb = xb_ref[i]
        xa_m, xa_p = shifted(xa)
        xb_m, xb_p = shifted(xb)
        acc1 = conv(sb1, (xa_m, xa, xa_p, xb_m, xb, xb_p), w1s_ref, 6 * C)
        h1 = jnp.maximum(acc1 + b1_ref[...], 0.0).astype(jnp.bfloat16)

        h1_m, h1_p = shifted(h1)
        acc2 = conv(sb2, (h1_m, h1, h1_p), w2s_ref, 3 * Cmid)
        y2 = jnp.maximum(acc2 + b2_ref[...], 0.0)
        yt = jnp.transpose(y2, (1, 0))
        o_ref[i] = yt.reshape(yt.shape[0], oh, ow)


def _fold_bn(conv_b, gamma, beta, rmean, rvar):
    scale = gamma / jnp.sqrt(rvar + _EPS)
    bias = beta + scale * (conv_b - rmean)
    return scale, bias


def kernel(x1, x2, up_w, up_b, c1_w, c1_b, bn1_g, bn1_b, bn1_m, bn1_v,
           c2_w, c2_b, bn2_g, bn2_b, bn2_m, bn2_v):
    N, C, H1, W1 = x1.shape
    oh, ow = 2 * H1, 2 * W1
    M = oh * ow
    Cmid = c1_w.shape[0]
    Cout = c2_w.shape[0]

    x1f = x1.reshape(N, C, H1 * W1)

    wup = jnp.transpose(up_w, (0, 2, 3, 1)).reshape(C, 4 * C)
    wup = wup.astype(jnp.bfloat16)
    bup = jnp.tile(up_b, 4)[None, :]

    up_out = pl.pallas_call(
        _upconv_kernel,
        out_shape=jax.ShapeDtypeStruct((N * H1, 2, W1, 2 * C),
                                       jnp.bfloat16),
        grid=(N // _UPB,),
        in_specs=[
            pl.BlockSpec((_UPB, C, H1 * W1), lambda n: (n, 0, 0)),
            pl.BlockSpec((C, 4 * C), lambda n: (0, 0)),
            pl.BlockSpec((1, 4 * C), lambda n: (0, 0)),
        ],
        out_specs=pl.BlockSpec((_UPB * H1, 2, W1, 2 * C),
                               lambda n: (n, 0, 0, 0)),
        compiler_params=pltpu.CompilerParams(
            dimension_semantics=("parallel",),
            vmem_limit_bytes=64 * 1024 * 1024,
        ),
    )(x1f, wup, bup)
    x1u = up_out.reshape(N, M, C)

    s1, b1 = _fold_bn(c1_b, bn1_g, bn1_b, bn1_m, bn1_v)
    s2, b2 = _fold_bn(c2_b, bn2_g, bn2_b, bn2_m, bn2_v)

    w1 = jnp.transpose(c1_w, (2, 3, 1, 0)).reshape(3, 3, 2 * C, Cmid)
    w1s = jnp.concatenate(
        [w1[:, 0, :C], w1[:, 1, :C], w1[:, 2, :C],
         w1[:, 0, C:], w1[:, 1, C:], w1[:, 2, C:]], axis=1)
    w1s = (w1s * s1[None, None, :]).astype(jnp.bfloat16)
    w2 = jnp.transpose(c2_w, (2, 3, 1, 0)).reshape(3, 3, Cmid, Cout)
    w2s = jnp.concatenate([w2[:, 0], w2[:, 1], w2[:, 2]], axis=1)
    w2s = (w2s * s2[None, None, :]).astype(jnp.bfloat16)

    body = functools.partial(_double_conv_kernel, oh=oh, ow=ow)
    out = pl.pallas_call(
        body,
        out_shape=jax.ShapeDtypeStruct((N, Cout, oh, ow), jnp.float32),
        grid=(N // _DCB,),
        in_specs=[
            pl.BlockSpec((_DCB, C, oh, ow), lambda n: (n, 0, 0, 0)),
            pl.BlockSpec((_DCB, M, C), lambda n: (n, 0, 0)),
            pl.BlockSpec((3, 6 * C, Cmid), lambda n: (0, 0, 0)),
            pl.BlockSpec((1, Cmid), lambda n: (0, 0)),
            pl.BlockSpec((3, 3 * Cmid, Cout), lambda n: (0, 0, 0)),
            pl.BlockSpec((1, Cout), lambda n: (0, 0)),
        ],
        out_specs=pl.BlockSpec((_DCB, Cout, oh, ow),
                               lambda n: (n, 0, 0, 0)),
        scratch_shapes=[
            pltpu.VMEM((M + 2 * ow, 6 * C), jnp.bfloat16),
            pltpu.VMEM((M + 2 * ow, 3 * Cmid), jnp.bfloat16),
            pltpu.VMEM((M + 2 * ow, 6 * C), jnp.bfloat16),
            pltpu.VMEM((M + 2 * ow, 3 * Cmid), jnp.bfloat16),
        ],
        compiler_params=pltpu.CompilerParams(
            dimension_semantics=("parallel",),
            vmem_limit_bytes=64 * 1024 * 1024,
        ),
    )(x2, x1u, w1s, b1[None, :], w2s, b2[None, :])

    return out
